# Initial kernel scaffold; baseline (speedup 1.0000x reference)
import jax
import jax.numpy as jnp
from jax.experimental import pallas as pl


def kernel(x_node, x_edge, edge_index, bn_node_gamma, bn_node_beta, node_W, node_b, bn_edge_gamma, bn_edge_beta, edge_W, edge_b, en1_W, en1_b, en_bn1_gamma, en_bn1_beta, en2_W, en2_b, en_bn2_gamma, en_bn2_beta, gru_Wih, gru_Whh, gru_bih, gru_bhh):
    raise NotImplementedError("write your pallas kernel here")


# speedup vs baseline: 1.0026x; 1.0026x over previous
import functools

import jax
import jax.numpy as jnp
from jax import lax
from jax.experimental import pallas as pl
from jax.experimental.pallas import tpu as pltpu
from jax.experimental.pallas import tpu_sc as plsc

N = 10000
NP = 10240
E = 160000
EP = 163840
F_NODE = 128
F_EDGE = 16
H = 16
EH = 64
N_LAYERS = 3
EPS = 1e-5

NC = 2
NS = 16
NW = NC * NS
EW = EP // NW
CH = 128
NCHUNK = EW // CH
NPS = NP // NS

_pallas_call = pl.pallas_call


@functools.cache
def _mesh():
    return plsc.VectorSubcoreMesh(
        core_axis_name="c", subcore_axis_name="s",
        num_cores=NC, num_subcores=NS)


def _node_stats_body(x_ref, o_ref):
    i = pl.program_id(0)
    x = x_ref[...]
    s = jnp.sum(x, axis=0)[None, :]
    q = jnp.sum(x * x, axis=0)[None, :]
    blk = jnp.concatenate([s, q, jnp.zeros((6, F_NODE), jnp.float32)], axis=0)

    @pl.when(i == 0)
    def _():
        o_ref[...] = blk

    @pl.when(i > 0)
    def _():
        o_ref[...] += blk


def _tc_node_stats(x_node):
    return _pallas_call(
        _node_stats_body,
        grid=(5,),
        in_specs=[pl.BlockSpec((2000, F_NODE), lambda i: (i, 0))],
        out_specs=pl.BlockSpec((8, F_NODE), lambda i: (0, 0)),
        out_shape=jax.ShapeDtypeStruct((8, F_NODE), jnp.float32),
    )(x_node)


def _edge_stats_body(x_ref, o_ref):
    i = pl.program_id(0)
    x = x_ref[...]
    xtx = lax.dot_general(x, x, (((0,), (0,)), ((), ())),
                          preferred_element_type=jnp.float32,
                 precision=lax.Precision.HIGHEST)
    s = jnp.sum(x, axis=0)[None, :]
    blk = jnp.concatenate([xtx, s, jnp.zeros((7, F_EDGE), jnp.float32)],
                          axis=0)

    @pl.when(i == 0)
    def _():
        o_ref[...] = blk

    @pl.when(i > 0)
    def _():
        o_ref[...] += blk


def _tc_edge_stats(x_edge):
    return _pallas_call(
        _edge_stats_body,
        grid=(40,),
        in_specs=[pl.BlockSpec((4000, F_EDGE), lambda i: (i, 0))],
        out_specs=pl.BlockSpec((24, F_EDGE), lambda i: (0, 0)),
        out_shape=jax.ShapeDtypeStruct((24, F_EDGE), jnp.float32),
    )(x_edge)


def _embed_body(x_ref, w_ref, b_ref, o_ref):
    o_ref[...] = jnp.dot(x_ref[...], w_ref[...],
                         preferred_element_type=jnp.float32,
                 precision=lax.Precision.HIGHEST) + b_ref[...]


def _tc_embed_node(x_node, WnT, bn):
    return _pallas_call(
        _embed_body,
        grid=(5,),
        in_specs=[pl.BlockSpec((2000, F_NODE), lambda i: (i, 0)),
                  pl.BlockSpec((F_NODE, H), lambda i: (0, 0)),
                  pl.BlockSpec((1, H), lambda i: (0, 0))],
        out_specs=pl.BlockSpec((2000, H), lambda i: (i, 0)),
        out_shape=jax.ShapeDtypeStruct((N, H), jnp.float32),
    )(x_node, WnT, bn)


def _amom_body(x_ref, WeT_ref, be_ref, W1T_ref, b1_ref, o_ref):
    i = pl.program_id(0)
    x = x_ref[...]
    h = jnp.dot(x, WeT_ref[...], preferred_element_type=jnp.float32,
                 precision=lax.Precision.HIGHEST) \
        + be_ref[...]
    z = jnp.dot(h, W1T_ref[...], preferred_element_type=jnp.float32,
                 precision=lax.Precision.HIGHEST) \
        + b1_ref[...]
    a = jnp.where(z >= 0, z, 0.8 * z)
    ata = lax.dot_general(a, a, (((0,), (0,)), ((), ())),
                          preferred_element_type=jnp.float32,
                 precision=lax.Precision.HIGHEST)
    s = jnp.sum(a, axis=0)[None, :]
    blk = jnp.concatenate([ata, s, jnp.zeros((7, EH), jnp.float32)], axis=0)

    @pl.when(i == 0)
    def _():
        o_ref[...] = blk

    @pl.when(i > 0)
    def _():
        o_ref[...] += blk


def _tc_amoments(x_edge, WeT, be, W1T, b1):
    return _pallas_call(
        _amom_body,
        grid=(40,),
        in_specs=[pl.BlockSpec((4000, F_EDGE), lambda i: (i, 0)),
                  pl.BlockSpec((F_EDGE, EH), lambda i: (0, 0)),
                  pl.BlockSpec((1, EH), lambda i: (0, 0)),
                  pl.BlockSpec((EH, EH), lambda i: (0, 0)),
                  pl.BlockSpec((1, EH), lambda i: (0, 0))],
        out_specs=pl.BlockSpec((72, EH), lambda i: (0, 0)),
        out_shape=jax.ShapeDtypeStruct((72, EH), jnp.float32),
    )(x_edge, WeT, be, W1T, b1)


def _msg_body(x_ref, hs_ref, WeT_ref, be_ref, W1T_ref, b1_ref, W2T_ref,
              d2_ref, o_ref):
    x = x_ref[...]
    h = jnp.dot(x, WeT_ref[...], preferred_element_type=jnp.float32,
                 precision=lax.Precision.HIGHEST) \
        + be_ref[...]
    z = jnp.dot(h, W1T_ref[...], preferred_element_type=jnp.float32,
                 precision=lax.Precision.HIGHEST) \
        + b1_ref[...]
    a = jnp.where(z >= 0, z, 0.8 * z)
    e2 = jnp.dot(a, W2T_ref[...], preferred_element_type=jnp.float32,
                 precision=lax.Precision.HIGHEST) \
        + d2_ref[...]
    hs = hs_ref[...]
    acc = hs[:, 0:1] * e2[:, 0:H]
    for i in range(1, H):
        acc = acc + hs[:, i:i + 1] * e2[:, H * i:H * (i + 1)]
    o_ref[...] = acc


def _tc_msg(x_edge, hs, WeT, be, W1T, b1, W2T, d2):
    return _pallas_call(
        _msg_body,
        grid=(40,),
        in_specs=[pl.BlockSpec((4096, F_EDGE), lambda i: (i, 0)),
                  pl.BlockSpec((4096, H), lambda i: (i, 0)),
                  pl.BlockSpec((F_EDGE, EH), lambda i: (0, 0)),
                  pl.BlockSpec((1, EH), lambda i: (0, 0)),
                  pl.BlockSpec((EH, EH), lambda i: (0, 0)),
                  pl.BlockSpec((1, EH), lambda i: (0, 0)),
                  pl.BlockSpec((EH, H * H), lambda i: (0, 0)),
                  pl.BlockSpec((1, H * H), lambda i: (0, 0))],
        out_specs=pl.BlockSpec((4096, H), lambda i: (i, 0)),
        out_shape=jax.ShapeDtypeStruct((EP, H), jnp.float32),
    )(x_edge, hs, WeT, be, W1T, b1, W2T, d2)


def _gru_body(sp_ref, cp_ref, h_ref, WihT_ref, WhhT_ref, bih_ref, bhh_ref,
              o_ref):
    s = sp_ref[0] + sp_ref[1]
    cnt = cp_ref[0] + cp_ref[1]
    m = s / jnp.maximum(cnt, 1.0)
    hp = h_ref[...]
    gi = jnp.dot(m, WihT_ref[...], preferred_element_type=jnp.float32,
                 precision=lax.Precision.HIGHEST) \
        + bih_ref[...]
    gh = jnp.dot(hp, WhhT_ref[...], preferred_element_type=jnp.float32,
                 precision=lax.Precision.HIGHEST) \
        + bhh_ref[...]
    r = jax.nn.sigmoid(gi[:, 0:H] + gh[:, 0:H])
    zg = jax.nn.sigmoid(gi[:, H:2 * H] + gh[:, H:2 * H])
    ng = jnp.tanh(gi[:, 2 * H:3 * H] + r * gh[:, 2 * H:3 * H])
    o_ref[...] = (1.0 - zg) * ng + zg * hp


def _tc_gru(sp, cp, h, WihT, WhhT, bih, bhh):
    return _pallas_call(
        _gru_body,
        grid=(5,),
        in_specs=[pl.BlockSpec((NC, 2000, H), lambda i: (0, i, 0)),
                  pl.BlockSpec((NC, 2000, H), lambda i: (0, i, 0)),
                  pl.BlockSpec((2000, H), lambda i: (i, 0)),
                  pl.BlockSpec((H, 3 * H), lambda i: (0, 0)),
                  pl.BlockSpec((H, 3 * H), lambda i: (0, 0)),
                  pl.BlockSpec((1, 3 * H), lambda i: (0, 0)),
                  pl.BlockSpec((1, 3 * H), lambda i: (0, 0))],
        out_specs=pl.BlockSpec((2000, H), lambda i: (i, 0)),
        out_shape=jax.ShapeDtypeStruct((N, H), jnp.float32),
    )(sp, cp, h, WihT, WhhT, bih, bhh)


def _sc_gather(h_nodes, src2):
    @functools.partial(
        pl.kernel, mesh=_mesh(),
        compiler_params=pltpu.CompilerParams(use_tc_tiling_on_sc=False),
        out_type=jax.ShapeDtypeStruct((EP, H), jnp.float32),
        scratch_types=[pltpu.VMEM((NCHUNK, CH), jnp.int32),
                       pltpu.VMEM((EW, H), jnp.float32),
                       pltpu.SemaphoreType.DMA],
    )
    def k(h_hbm, src_hbm, out_hbm, idx_v, rows_v, sem):
        c = lax.axis_index("c")
        s = lax.axis_index("s")
        wid = s * NC + c
        pltpu.sync_copy(src_hbm.at[pl.ds(wid * NCHUNK, NCHUNK)], idx_v)

        def body(j, _):
            pltpu.async_copy(h_hbm.at[idx_v.at[j]],
                             rows_v.at[pl.ds(j * CH, CH)], sem).wait()
            return 0

        lax.fori_loop(0, NCHUNK, body, 0)
        pltpu.sync_copy(rows_v, out_hbm.at[pl.ds(wid * EW, EW)])

    return k(h_nodes, src2)


def _sc_scatter(msg, dst2, zeros_n):
    @functools.partial(
        pl.kernel, mesh=_mesh(),
        compiler_params=pltpu.CompilerParams(use_tc_tiling_on_sc=False),
        out_type=jax.ShapeDtypeStruct((NC, NP, H), jnp.float32),
        scratch_types=[pltpu.VMEM((NCHUNK, CH), jnp.int32),
                       pltpu.VMEM((EW, H), jnp.float32),
                       pltpu.VMEM_SHARED((NP, H), jnp.float32),
                       pltpu.SemaphoreType.DMA],
    )
    def k(msg_hbm, dst_hbm, zero_hbm, out_hbm, idx_v, msg_v, acc_sh, sem):
        c = lax.axis_index("c")
        s = lax.axis_index("s")
        wid = s * NC + c
        pltpu.sync_copy(zero_hbm.at[pl.ds(s * NPS, NPS)],
                        acc_sh.at[pl.ds(s * NPS, NPS)])
        pltpu.sync_copy(dst_hbm.at[pl.ds(wid * NCHUNK, NCHUNK)], idx_v)
        pltpu.sync_copy(msg_hbm.at[pl.ds(wid * EW, EW)], msg_v)
        plsc.subcore_barrier()

        def body(j, _):
            pltpu.async_copy(msg_v.at[pl.ds(j * CH, CH)],
                             acc_sh.at[idx_v.at[j]], sem, add=True).wait()
            return 0

        lax.fori_loop(0, NCHUNK, body, 0)
        plsc.subcore_barrier()
        pltpu.sync_copy(acc_sh.at[pl.ds(s * NPS, NPS)],
                        out_hbm.at[c, pl.ds(s * NPS, NPS)])

    return k(msg, dst2, zeros_n)


def _sc_count(dst2, ones_c, zeros_n):
    @functools.partial(
        pl.kernel, mesh=_mesh(),
        compiler_params=pltpu.CompilerParams(use_tc_tiling_on_sc=False),
        out_type=jax.ShapeDtypeStruct((NC, NP, H), jnp.float32),
        scratch_types=[pltpu.VMEM((NCHUNK, CH), jnp.int32),
                       pltpu.VMEM((CH, H), jnp.float32),
                       pltpu.VMEM_SHARED((NP, H), jnp.float32),
                       pltpu.SemaphoreType.DMA],
    )
    def k(dst_hbm, ones_hbm, zero_hbm, out_hbm, idx_v, ones_v, acc_sh, sem):
        c = lax.axis_index("c")
        s = lax.axis_index("s")
        wid = s * NC + c
        pltpu.sync_copy(zero_hbm.at[pl.ds(s * NPS, NPS)],
                        acc_sh.at[pl.ds(s * NPS, NPS)])
        pltpu.sync_copy(dst_hbm.at[pl.ds(wid * NCHUNK, NCHUNK)], idx_v)
        pltpu.sync_copy(ones_hbm, ones_v)
        plsc.subcore_barrier()

        def body(j, _):
            pltpu.async_copy(ones_v, acc_sh.at[idx_v.at[j]], sem,
                             add=True).wait()
            return 0

        lax.fori_loop(0, NCHUNK, body, 0)
        plsc.subcore_barrier()
        pltpu.sync_copy(acc_sh.at[pl.ds(s * NPS, NPS)],
                        out_hbm.at[c, pl.ds(s * NPS, NPS)])

    return k(dst2, ones_c, zeros_n)


def kernel(x_node, x_edge, edge_index, bn_node_gamma, bn_node_beta, node_W,
           node_b, bn_edge_gamma, bn_edge_beta, edge_W, edge_b, en1_W, en1_b,
           en_bn1_gamma, en_bn1_beta, en2_W, en2_b, en_bn2_gamma, en_bn2_beta,
           gru_Wih, gru_Whh, gru_bih, gru_bhh):
    f32 = jnp.float32
    pad = EP - E
    src2 = jnp.concatenate(
        [edge_index[0], jnp.zeros((pad,), jnp.int32)]).reshape(EP // CH, CH)
    dst2 = jnp.concatenate(
        [edge_index[1], jnp.full((pad,), N, jnp.int32)]).reshape(EP // CH, CH)
    x_edge_p = jnp.concatenate(
        [x_edge, jnp.zeros((pad, F_EDGE), f32)], axis=0)

    nstat = _tc_node_stats(x_node)
    mu_n = nstat[0] / N
    var_n = nstat[1] / N - mu_n * mu_n
    g_n = bn_node_gamma / jnp.sqrt(var_n + EPS)
    c_n = bn_node_beta - mu_n * g_n
    WnT = (node_W * g_n[None, :]).T
    bn = (c_n @ node_W.T + node_b)[None, :]
    h0 = _tc_embed_node(x_node, WnT, bn)

    estat = _tc_edge_stats(x_edge)
    exx = estat[0:F_EDGE]
    mu_e = estat[F_EDGE] / E
    cov_e = exx / E - mu_e[:, None] * mu_e[None, :]
    g_e = bn_edge_gamma / jnp.sqrt(jnp.diag(cov_e) + EPS)
    c_e = bn_edge_beta - mu_e * g_e
    We = edge_W * g_e[None, :]
    be_v = c_e @ edge_W.T + edge_b
    mean_h = mu_e @ We.T + be_v
    cov_h = We @ cov_e @ We.T
    mean_z1 = mean_h @ en1_W.T + en1_b
    var_z1 = jnp.sum((en1_W @ cov_h) * en1_W, axis=1)
    g1 = en_bn1_gamma / jnp.sqrt(var_z1 + EPS)
    c1 = en_bn1_beta - mean_z1 * g1
    W1 = en1_W * g1[:, None]
    b1_v = en1_b * g1 + c1

    WeT = We.T
    be = be_v[None, :]
    W1T = W1.T
    b1 = b1_v[None, :]

    amom = _tc_amoments(x_edge, WeT, be, W1T, b1)
    ata = amom[0:EH]
    mu_a = amom[EH] / E
    cov_a = ata / E - mu_a[:, None] * mu_a[None, :]
    mean_z2 = mu_a @ en2_W.T + en2_b
    var_z2 = jnp.sum((en2_W @ cov_a) * en2_W, axis=1)
    g2 = en_bn2_gamma / jnp.sqrt(var_z2 + EPS)
    W2T = (en2_W * g2[:, None]).T
    d2 = (en_bn2_beta + (en2_b - mean_z2) * g2)[None, :]

    WihT = gru_Wih.T
    WhhT = gru_Whh.T
    bih = gru_bih[None, :]
    bhh = gru_bhh[None, :]

    zeros_n = jnp.zeros((NP, H), f32)
    ones_c = jnp.ones((CH, H), f32)
    cp = _sc_count(dst2, ones_c, zeros_n)

    h = h0
    for _ in range(N_LAYERS):
        hs = _sc_gather(h, src2)
        msg = _tc_msg(x_edge_p, hs, WeT, be, W1T, b1, W2T, d2)
        sp = _sc_scatter(msg, dst2, zeros_n)
        h = _tc_gru(sp, cp, h, WihT, WhhT, bih, bhh)
    return h


# speedup vs baseline: 1.0094x; 1.0067x over previous
import functools

import jax
import jax.numpy as jnp
from jax import lax
from jax.experimental import pallas as pl
from jax.experimental.pallas import tpu as pltpu
from jax.experimental.pallas import tpu_sc as plsc

N = 10000
NP = 10240
E = 160000
EP = 163840
F_NODE = 128
F_EDGE = 16
H = 16
EH = 64
N_LAYERS = 3
EPS = 1e-5

NC = 2
NS = 16
NW = NC * NS
EW = EP // NW
CH = 128
NCHUNK = EW // CH
NPS = NP // NS

_pallas_call = pl.pallas_call


@functools.cache
def _mesh():
    return plsc.VectorSubcoreMesh(
        core_axis_name="c", subcore_axis_name="s",
        num_cores=NC, num_subcores=NS)


def _node_stats_body(x_ref, o_ref):
    i = pl.program_id(0)
    x = x_ref[...]
    s = jnp.sum(x, axis=0)[None, :]
    q = jnp.sum(x * x, axis=0)[None, :]
    blk = jnp.concatenate([s, q, jnp.zeros((6, F_NODE), jnp.float32)], axis=0)

    @pl.when(i == 0)
    def _():
        o_ref[...] = blk

    @pl.when(i > 0)
    def _():
        o_ref[...] += blk


def _tc_node_stats(x_node):
    return _pallas_call(
        _node_stats_body,
        grid=(5,),
        in_specs=[pl.BlockSpec((2000, F_NODE), lambda i: (i, 0))],
        out_specs=pl.BlockSpec((8, F_NODE), lambda i: (0, 0)),
        out_shape=jax.ShapeDtypeStruct((8, F_NODE), jnp.float32),
    )(x_node)


def _edge_stats_body(x_ref, o_ref):
    i = pl.program_id(0)
    x = x_ref[...]
    xtx = lax.dot_general(x, x, (((0,), (0,)), ((), ())),
                          preferred_element_type=jnp.float32,
                 precision=lax.Precision.HIGHEST)
    s = jnp.sum(x, axis=0)[None, :]
    blk = jnp.concatenate([xtx, s, jnp.zeros((7, F_EDGE), jnp.float32)],
                          axis=0)

    @pl.when(i == 0)
    def _():
        o_ref[...] = blk

    @pl.when(i > 0)
    def _():
        o_ref[...] += blk


def _tc_edge_stats(x_edge):
    return _pallas_call(
        _edge_stats_body,
        grid=(40,),
        in_specs=[pl.BlockSpec((4000, F_EDGE), lambda i: (i, 0))],
        out_specs=pl.BlockSpec((24, F_EDGE), lambda i: (0, 0)),
        out_shape=jax.ShapeDtypeStruct((24, F_EDGE), jnp.float32),
    )(x_edge)


def _embed_body(x_ref, w_ref, b_ref, o_ref):
    o_ref[...] = jnp.dot(x_ref[...], w_ref[...],
                         preferred_element_type=jnp.float32,
                 precision=lax.Precision.HIGHEST) + b_ref[...]


def _tc_embed_node(x_node, WnT, bn):
    return _pallas_call(
        _embed_body,
        grid=(5,),
        in_specs=[pl.BlockSpec((2000, F_NODE), lambda i: (i, 0)),
                  pl.BlockSpec((F_NODE, H), lambda i: (0, 0)),
                  pl.BlockSpec((1, H), lambda i: (0, 0))],
        out_specs=pl.BlockSpec((2000, H), lambda i: (i, 0)),
        out_shape=jax.ShapeDtypeStruct((N, H), jnp.float32),
    )(x_node, WnT, bn)


def _amom_body(x_ref, WeT_ref, be_ref, W1T_ref, b1_ref, o_ref):
    i = pl.program_id(0)
    x = x_ref[...]
    h = jnp.dot(x, WeT_ref[...], preferred_element_type=jnp.float32,
                 precision=lax.Precision.HIGHEST) \
        + be_ref[...]
    z = jnp.dot(h, W1T_ref[...], preferred_element_type=jnp.float32,
                 precision=lax.Precision.HIGHEST) \
        + b1_ref[...]
    a = jnp.where(z >= 0, z, 0.8 * z)
    ata = lax.dot_general(a, a, (((0,), (0,)), ((), ())),
                          preferred_element_type=jnp.float32,
                 precision=lax.Precision.HIGHEST)
    s = jnp.sum(a, axis=0)[None, :]
    blk = jnp.concatenate([ata, s, jnp.zeros((7, EH), jnp.float32)], axis=0)

    @pl.when(i == 0)
    def _():
        o_ref[...] = blk

    @pl.when(i > 0)
    def _():
        o_ref[...] += blk


def _tc_amoments(x_edge, WeT, be, W1T, b1):
    return _pallas_call(
        _amom_body,
        grid=(40,),
        in_specs=[pl.BlockSpec((4000, F_EDGE), lambda i: (i, 0)),
                  pl.BlockSpec((F_EDGE, EH), lambda i: (0, 0)),
                  pl.BlockSpec((1, EH), lambda i: (0, 0)),
                  pl.BlockSpec((EH, EH), lambda i: (0, 0)),
                  pl.BlockSpec((1, EH), lambda i: (0, 0))],
        out_specs=pl.BlockSpec((72, EH), lambda i: (0, 0)),
        out_shape=jax.ShapeDtypeStruct((72, EH), jnp.float32),
    )(x_edge, WeT, be, W1T, b1)


def _e2_body(x_ref, WeT_ref, be_ref, W1T_ref, b1_ref, W2T_ref,
             d2_ref, o_ref):
    x = x_ref[...]
    h = jnp.dot(x, WeT_ref[...], preferred_element_type=jnp.float32,
                precision=lax.Precision.HIGHEST) \
        + be_ref[...]
    z = jnp.dot(h, W1T_ref[...], preferred_element_type=jnp.float32,
                precision=lax.Precision.HIGHEST) \
        + b1_ref[...]
    a = jnp.where(z >= 0, z, 0.8 * z)
    o_ref[...] = jnp.dot(a, W2T_ref[...], preferred_element_type=jnp.float32,
                         precision=lax.Precision.HIGHEST) \
        + d2_ref[...]


def _tc_e2(x_edge, WeT, be, W1T, b1, W2T, d2):
    return _pallas_call(
        _e2_body,
        grid=(40,),
        in_specs=[pl.BlockSpec((4096, F_EDGE), lambda i: (i, 0)),
                  pl.BlockSpec((F_EDGE, EH), lambda i: (0, 0)),
                  pl.BlockSpec((1, EH), lambda i: (0, 0)),
                  pl.BlockSpec((EH, EH), lambda i: (0, 0)),
                  pl.BlockSpec((1, EH), lambda i: (0, 0)),
                  pl.BlockSpec((EH, H * H), lambda i: (0, 0)),
                  pl.BlockSpec((1, H * H), lambda i: (0, 0))],
        out_specs=pl.BlockSpec((4096, H * H), lambda i: (i, 0)),
        out_shape=jax.ShapeDtypeStruct((EP, H * H), jnp.float32),
    )(x_edge, WeT, be, W1T, b1, W2T, d2)


def _msg_body(e2_ref, hs_ref, o_ref):
    e2 = e2_ref[...]
    hs = hs_ref[...]
    acc = hs[:, 0:1] * e2[:, 0:H]
    for i in range(1, H):
        acc = acc + hs[:, i:i + 1] * e2[:, H * i:H * (i + 1)]
    o_ref[...] = acc


def _tc_msg(e2, hs):
    return _pallas_call(
        _msg_body,
        grid=(40,),
        in_specs=[pl.BlockSpec((4096, H * H), lambda i: (i, 0)),
                  pl.BlockSpec((4096, H), lambda i: (i, 0))],
        out_specs=pl.BlockSpec((4096, H), lambda i: (i, 0)),
        out_shape=jax.ShapeDtypeStruct((EP, H), jnp.float32),
    )(e2, hs)


def _gru_body(sp_ref, cp_ref, h_ref, WihT_ref, WhhT_ref, bih_ref, bhh_ref,
              o_ref):
    s = sp_ref[0] + sp_ref[1]
    cnt = cp_ref[0] + cp_ref[1]
    m = s / jnp.maximum(cnt, 1.0)
    hp = h_ref[...]
    gi = jnp.dot(m, WihT_ref[...], preferred_element_type=jnp.float32,
                 precision=lax.Precision.HIGHEST) \
        + bih_ref[...]
    gh = jnp.dot(hp, WhhT_ref[...], preferred_element_type=jnp.float32,
                 precision=lax.Precision.HIGHEST) \
        + bhh_ref[...]
    r = jax.nn.sigmoid(gi[:, 0:H] + gh[:, 0:H])
    zg = jax.nn.sigmoid(gi[:, H:2 * H] + gh[:, H:2 * H])
    ng = jnp.tanh(gi[:, 2 * H:3 * H] + r * gh[:, 2 * H:3 * H])
    o_ref[...] = (1.0 - zg) * ng + zg * hp


def _tc_gru(sp, cp, h, WihT, WhhT, bih, bhh):
    return _pallas_call(
        _gru_body,
        grid=(5,),
        in_specs=[pl.BlockSpec((NC, 2000, H), lambda i: (0, i, 0)),
                  pl.BlockSpec((NC, 2000, H), lambda i: (0, i, 0)),
                  pl.BlockSpec((2000, H), lambda i: (i, 0)),
                  pl.BlockSpec((H, 3 * H), lambda i: (0, 0)),
                  pl.BlockSpec((H, 3 * H), lambda i: (0, 0)),
                  pl.BlockSpec((1, 3 * H), lambda i: (0, 0)),
                  pl.BlockSpec((1, 3 * H), lambda i: (0, 0))],
        out_specs=pl.BlockSpec((2000, H), lambda i: (i, 0)),
        out_shape=jax.ShapeDtypeStruct((N, H), jnp.float32),
    )(sp, cp, h, WihT, WhhT, bih, bhh)


def _sc_gather(h_nodes, src2):
    @functools.partial(
        pl.kernel, mesh=_mesh(),
        compiler_params=pltpu.CompilerParams(use_tc_tiling_on_sc=False),
        out_type=jax.ShapeDtypeStruct((EP, H), jnp.float32),
        scratch_types=[pltpu.VMEM((NCHUNK, CH), jnp.int32),
                       pltpu.VMEM((EW, H), jnp.float32),
                       pltpu.SemaphoreType.DMA],
    )
    def k(h_hbm, src_hbm, out_hbm, idx_v, rows_v, sem):
        c = lax.axis_index("c")
        s = lax.axis_index("s")
        wid = s * NC + c
        pltpu.sync_copy(src_hbm.at[pl.ds(wid * NCHUNK, NCHUNK)], idx_v)

        def body(j, _):
            pltpu.async_copy(h_hbm.at[idx_v.at[j]],
                             rows_v.at[pl.ds(j * CH, CH)], sem).wait()
            return 0

        lax.fori_loop(0, NCHUNK, body, 0)
        pltpu.sync_copy(rows_v, out_hbm.at[pl.ds(wid * EW, EW)])

    return k(h_nodes, src2)


def _sc_scatter(msg, dst2, zeros_n):
    @functools.partial(
        pl.kernel, mesh=_mesh(),
        compiler_params=pltpu.CompilerParams(use_tc_tiling_on_sc=False),
        out_type=jax.ShapeDtypeStruct((NC, NP, H), jnp.float32),
        scratch_types=[pltpu.VMEM((NCHUNK, CH), jnp.int32),
                       pltpu.VMEM((EW, H), jnp.float32),
                       pltpu.VMEM_SHARED((NP, H), jnp.float32),
                       pltpu.SemaphoreType.DMA],
    )
    def k(msg_hbm, dst_hbm, zero_hbm, out_hbm, idx_v, msg_v, acc_sh, sem):
        c = lax.axis_index("c")
        s = lax.axis_index("s")
        wid = s * NC + c
        pltpu.sync_copy(zero_hbm.at[pl.ds(s * NPS, NPS)],
                        acc_sh.at[pl.ds(s * NPS, NPS)])
        pltpu.sync_copy(dst_hbm.at[pl.ds(wid * NCHUNK, NCHUNK)], idx_v)
        pltpu.sync_copy(msg_hbm.at[pl.ds(wid * EW, EW)], msg_v)
        plsc.subcore_barrier()

        def body(j, _):
            pltpu.async_copy(msg_v.at[pl.ds(j * CH, CH)],
                             acc_sh.at[idx_v.at[j]], sem, add=True).wait()
            return 0

        lax.fori_loop(0, NCHUNK, body, 0)
        plsc.subcore_barrier()
        pltpu.sync_copy(acc_sh.at[pl.ds(s * NPS, NPS)],
                        out_hbm.at[c, pl.ds(s * NPS, NPS)])

    return k(msg, dst2, zeros_n)


def _sc_count(dst2, ones_c, zeros_n):
    @functools.partial(
        pl.kernel, mesh=_mesh(),
        compiler_params=pltpu.CompilerParams(use_tc_tiling_on_sc=False),
        out_type=jax.ShapeDtypeStruct((NC, NP, H), jnp.float32),
        scratch_types=[pltpu.VMEM((NCHUNK, CH), jnp.int32),
                       pltpu.VMEM((CH, H), jnp.float32),
                       pltpu.VMEM_SHARED((NP, H), jnp.float32),
                       pltpu.SemaphoreType.DMA],
    )
    def k(dst_hbm, ones_hbm, zero_hbm, out_hbm, idx_v, ones_v, acc_sh, sem):
        c = lax.axis_index("c")
        s = lax.axis_index("s")
        wid = s * NC + c
        pltpu.sync_copy(zero_hbm.at[pl.ds(s * NPS, NPS)],
                        acc_sh.at[pl.ds(s * NPS, NPS)])
        pltpu.sync_copy(dst_hbm.at[pl.ds(wid * NCHUNK, NCHUNK)], idx_v)
        pltpu.sync_copy(ones_hbm, ones_v)
        plsc.subcore_barrier()

        def body(j, _):
            pltpu.async_copy(ones_v, acc_sh.at[idx_v.at[j]], sem,
                             add=True).wait()
            return 0

        lax.fori_loop(0, NCHUNK, body, 0)
        plsc.subcore_barrier()
        pltpu.sync_copy(acc_sh.at[pl.ds(s * NPS, NPS)],
                        out_hbm.at[c, pl.ds(s * NPS, NPS)])

    return k(dst2, ones_c, zeros_n)


def kernel(x_node, x_edge, edge_index, bn_node_gamma, bn_node_beta, node_W,
           node_b, bn_edge_gamma, bn_edge_beta, edge_W, edge_b, en1_W, en1_b,
           en_bn1_gamma, en_bn1_beta, en2_W, en2_b, en_bn2_gamma, en_bn2_beta,
           gru_Wih, gru_Whh, gru_bih, gru_bhh):
    f32 = jnp.float32
    pad = EP - E
    src2 = jnp.concatenate(
        [edge_index[0], jnp.zeros((pad,), jnp.int32)]).reshape(EP // CH, CH)
    dst2 = jnp.concatenate(
        [edge_index[1], jnp.full((pad,), N, jnp.int32)]).reshape(EP // CH, CH)
    x_edge_p = jnp.concatenate(
        [x_edge, jnp.zeros((pad, F_EDGE), f32)], axis=0)

    nstat = _tc_node_stats(x_node)
    mu_n = nstat[0] / N
    var_n = nstat[1] / N - mu_n * mu_n
    g_n = bn_node_gamma / jnp.sqrt(var_n + EPS)
    c_n = bn_node_beta - mu_n * g_n
    WnT = (node_W * g_n[None, :]).T
    bn = (c_n @ node_W.T + node_b)[None, :]
    h0 = _tc_embed_node(x_node, WnT, bn)

    estat = _tc_edge_stats(x_edge)
    exx = estat[0:F_EDGE]
    mu_e = estat[F_EDGE] / E
    cov_e = exx / E - mu_e[:, None] * mu_e[None, :]
    g_e = bn_edge_gamma / jnp.sqrt(jnp.diag(cov_e) + EPS)
    c_e = bn_edge_beta - mu_e * g_e
    We = edge_W * g_e[None, :]
    be_v = c_e @ edge_W.T + edge_b
    mean_h = mu_e @ We.T + be_v
    cov_h = We @ cov_e @ We.T
    mean_z1 = mean_h @ en1_W.T + en1_b
    var_z1 = jnp.sum((en1_W @ cov_h) * en1_W, axis=1)
    g1 = en_bn1_gamma / jnp.sqrt(var_z1 + EPS)
    c1 = en_bn1_beta - mean_z1 * g1
    W1 = en1_W * g1[:, None]
    b1_v = en1_b * g1 + c1

    WeT = We.T
    be = be_v[None, :]
    W1T = W1.T
    b1 = b1_v[None, :]

    amom = _tc_amoments(x_edge, WeT, be, W1T, b1)
    ata = amom[0:EH]
    mu_a = amom[EH] / E
    cov_a = ata / E - mu_a[:, None] * mu_a[None, :]
    mean_z2 = mu_a @ en2_W.T + en2_b
    var_z2 = jnp.sum((en2_W @ cov_a) * en2_W, axis=1)
    g2 = en_bn2_gamma / jnp.sqrt(var_z2 + EPS)
    W2T = (en2_W * g2[:, None]).T
    d2 = (en_bn2_beta + (en2_b - mean_z2) * g2)[None, :]

    WihT = gru_Wih.T
    WhhT = gru_Whh.T
    bih = gru_bih[None, :]
    bhh = gru_bhh[None, :]

    zeros_n = jnp.zeros((NP, H), f32)
    ones_c = jnp.ones((CH, H), f32)
    cp = _sc_count(dst2, ones_c, zeros_n)

    e2 = _tc_e2(x_edge_p, WeT, be, W1T, b1, W2T, d2)

    h = h0
    for _ in range(N_LAYERS):
        hs = _sc_gather(h, src2)
        msg = _tc_msg(e2, hs)
        sp = _sc_scatter(msg, dst2, zeros_n)
        h = _tc_gru(sp, cp, h, WihT, WhhT, bih, bhh)
    return h
